# baseline (speedup 1.0000x reference)
import functools

import jax
import jax.numpy as jnp
from jax import lax
from jax.experimental import pallas as pl
from jax.experimental.pallas import tpu as pltpu
from jax.experimental.pallas import tpu_sc as plsc

N = 10000
E = 320000
D = 128
DOUT = 121

NC, NS = 2, 16
NW = NC * NS
NPAD = 10240
EW = 10240
EPAD = NW * EW
CH = 128
NCH = EW // CH
WIN = 8
RPT = NPAD // NS
BIGC = 1
NCHB = 160
NCHS = 0
TCH = EPAD // CH
BR = 640


def _deg_body(srcp, dstp, ones_h, zeros_h, out, sidx, didx, ob, acc_s, acc_d):
    c = lax.axis_index("c")
    s = lax.axis_index("s")
    wid = c * NS + s
    pltpu.sync_copy(ones_h, ob)
    pltpu.sync_copy(zeros_h, acc_s.at[pl.ds(s * RPT, RPT)])
    pltpu.sync_copy(zeros_h, acc_d.at[pl.ds(s * RPT, RPT)])
    pltpu.sync_copy(srcp.at[pl.ds(wid * NCH, NCH)], sidx)
    pltpu.sync_copy(dstp.at[pl.ds(wid * NCH, NCH)], didx)
    plsc.subcore_barrier()

    def chunk(j, carry):
        pltpu.sync_copy(ob, acc_s.at[sidx.at[j]], add=True)
        pltpu.sync_copy(ob, acc_d.at[didx.at[j]], add=True)
        return carry

    lax.fori_loop(0, NCH, chunk, 0)
    plsc.subcore_barrier()
    pltpu.sync_copy(acc_s.at[pl.ds(s * RPT, RPT)],
                    out.at[c, 0, pl.ds(s * RPT, RPT)])
    pltpu.sync_copy(acc_d.at[pl.ds(s * RPT, RPT)],
                    out.at[c, 1, pl.ds(s * RPT, RPT)])


_deg_call = pl.kernel(
    _deg_body,
    out_type=jax.ShapeDtypeStruct((NC, 2, NPAD), jnp.float32),
    mesh=plsc.VectorSubcoreMesh(core_axis_name="c", subcore_axis_name="s"),
    scratch_types=[
        pltpu.VMEM((NCH, CH), jnp.int32),
        pltpu.VMEM((NCH, CH), jnp.int32),
        pltpu.VMEM((CH,), jnp.float32),
        pltpu.VMEM_SHARED((NPAD,), jnp.float32),
        pltpu.VMEM_SHARED((NPAD,), jnp.float32),
    ],
)


def _agg_body(xs, srcp, dstp, out, sbuf, dbuf, rows, acc, gsem, ssem):
    c = lax.axis_index("c")
    s = lax.axis_index("s")
    big = c == BIGC
    nch = jnp.where(big, NCHB, NCHS)
    cb = jnp.where(big, s * NCHB, jnp.minimum(NS * NCHB + s * NCHS, TCH - WIN))

    def zrow(r, carry):
        for k in range(D // 16):
            rows[0, r, pl.ds(k * 16, 16)] = jnp.zeros((16,), jnp.float32)
        return carry

    lax.fori_loop(0, CH, zrow, 0)
    for t in range(RPT // CH):
        pltpu.sync_copy(rows.at[0], acc.at[pl.ds(s * RPT + t * CH, CH)])
    plsc.subcore_barrier()

    def refill(half, w):
        pltpu.sync_copy(srcp.at[pl.ds(cb + w * WIN, WIN)], sbuf.at[half])
        pltpu.sync_copy(dstp.at[pl.ds(cb + w * WIN, WIN)], dbuf.at[half])

    def wait_gather(p):
        pltpu.make_async_copy(xs.at[sbuf.at[0, 0]], rows.at[p], gsem).wait()

    def wait_scatter(q):
        pltpu.make_async_copy(rows.at[q], acc.at[dbuf.at[0, 0]], ssem).wait()

    refill(0, 0)

    @pl.when(nch > 0)
    def _():
        pltpu.async_copy(xs.at[sbuf.at[0, 0]], rows.at[0], gsem)

    def body(i, carry):
        base0 = 2 * WIN * i
        @pl.when(i > 0)
        def _():
            wait_scatter(1)
        refill(1, 2 * i + 1)
        for j in range(2 * WIN):
            base = base0 + j
            p = j % 2
            q = 1 - p
            h, jj = (0, j) if j < WIN else (1, j - WIN)
            wait_gather(p)
            if j > 0:
                wait_scatter(q)
            if j == WIN:
                @pl.when(base0 + 2 * WIN < nch)
                def _():
                    refill(0, 2 * i + 2)
            nh, njj = (0, 0) if j == 2 * WIN - 1 else \
                      ((0, j + 1) if j + 1 < WIN else (1, j + 1 - WIN))

            @pl.when(base + 1 < nch)
            def _():
                pltpu.async_copy(xs.at[sbuf.at[nh, njj]], rows.at[q], gsem)

            pltpu.async_copy(rows.at[p], acc.at[dbuf.at[h, jj]], ssem,
                             add=True)
        return carry

    lax.fori_loop(0, nch // (2 * WIN), body, 0)

    @pl.when(nch > 0)
    def _():
        wait_scatter(1)

    plsc.subcore_barrier()
    pltpu.sync_copy(acc.at[pl.ds(s * RPT, RPT)],
                    out.at[c, pl.ds(s * RPT, RPT)])


_agg_call = pl.kernel(
    _agg_body,
    out_type=jax.ShapeDtypeStruct((NC, NPAD, D), jnp.float32),
    mesh=plsc.VectorSubcoreMesh(core_axis_name="c", subcore_axis_name="s"),
    scratch_types=[
        pltpu.VMEM((2, WIN, CH), jnp.int32),
        pltpu.VMEM((2, WIN, CH), jnp.int32),
        pltpu.VMEM((2, CH, D), jnp.float32),
        pltpu.VMEM_SHARED((NPAD, D), jnp.float32),
        pltpu.SemaphoreType.DMA,
        pltpu.SemaphoreType.DMA,
    ],
)


def _scale_body(feat_ref, cnt_ref, xs_ref, ns_ref, nd_ref):
    cnt = cnt_ref[...]
    cs = cnt[0, 0] + cnt[1, 0]
    cd = cnt[0, 1] + cnt[1, 1]
    ns = lax.rsqrt(jnp.maximum(cs, 1.0))
    nd = lax.rsqrt(jnp.maximum(cd, 1.0))
    xs_ref[...] = feat_ref[...] * ns
    ns_ref[...] = ns
    nd_ref[...] = nd


_scale_call = pl.pallas_call(
    _scale_body,
    grid=(NPAD // BR,),
    in_specs=[
        pl.BlockSpec((BR, D), lambda i: (i, 0)),
        pl.BlockSpec((NC, 2, BR, 1), lambda i: (0, 0, i, 0)),
    ],
    out_specs=[
        pl.BlockSpec((BR, D), lambda i: (i, 0)),
        pl.BlockSpec((BR, 1), lambda i: (i, 0)),
        pl.BlockSpec((BR, 1), lambda i: (i, 0)),
    ],
    out_shape=[
        jax.ShapeDtypeStruct((NPAD, D), jnp.float32),
        jax.ShapeDtypeStruct((NPAD, 1), jnp.float32),
        jax.ShapeDtypeStruct((NPAD, 1), jnp.float32),
    ],
)


def _mm1_body(p_ref, w_ref, b_ref, nd_ref, ns_ref, o_ref):
    a = p_ref[0] + p_ref[1]
    h = jnp.dot(a, w_ref[...], preferred_element_type=jnp.float32)
    h = h * nd_ref[...] + b_ref[...]
    o_ref[...] = jnp.maximum(h, 0.0) * ns_ref[...]


_mm1_call = pl.pallas_call(
    _mm1_body,
    grid=(NPAD // BR,),
    in_specs=[
        pl.BlockSpec((NC, BR, D), lambda i: (0, i, 0)),
        pl.BlockSpec((D, D), lambda i: (0, 0)),
        pl.BlockSpec((1, D), lambda i: (0, 0)),
        pl.BlockSpec((BR, 1), lambda i: (i, 0)),
        pl.BlockSpec((BR, 1), lambda i: (i, 0)),
    ],
    out_specs=pl.BlockSpec((BR, D), lambda i: (i, 0)),
    out_shape=jax.ShapeDtypeStruct((NPAD, D), jnp.float32),
)


def _mm2_body(p_ref, w_ref, b_ref, nd_ref, o_ref):
    a = p_ref[0] + p_ref[1]
    h = jnp.dot(a, w_ref[...], preferred_element_type=jnp.float32)
    h = h * nd_ref[...] + b_ref[...]
    o_ref[...] = jax.nn.sigmoid(h)


_mm2_call = pl.pallas_call(
    _mm2_body,
    grid=(NPAD // BR,),
    in_specs=[
        pl.BlockSpec((NC, BR, D), lambda i: (0, i, 0)),
        pl.BlockSpec((D, D), lambda i: (0, 0)),
        pl.BlockSpec((1, D), lambda i: (0, 0)),
        pl.BlockSpec((BR, 1), lambda i: (i, 0)),
    ],
    out_specs=pl.BlockSpec((BR, D), lambda i: (i, 0)),
    out_shape=jax.ShapeDtypeStruct((NPAD, D), jnp.float32),
)


@jax.jit
def kernel(feat, edge_index, W1, b1, W2, b2):
    src = edge_index[0].astype(jnp.int32)
    dst = edge_index[1].astype(jnp.int32)
    filler = jnp.full((EPAD - E,), N, jnp.int32)
    srcp = jnp.concatenate([src, filler]).reshape(TCH, CH)
    dstp = jnp.concatenate([dst, filler]).reshape(TCH, CH)
    featp = jnp.pad(feat, ((0, NPAD - N), (0, 0)))
    ones_h = jnp.ones((CH,), jnp.float32)
    zeros_h = jnp.zeros((RPT,), jnp.float32)

    cnt = _deg_call(srcp, dstp, ones_h, zeros_h).reshape(NC, 2, NPAD, 1)
    xs, ns, nd = _scale_call(featp, cnt)
    p1 = _agg_call(xs, srcp, dstp)
    h1s = _mm1_call(p1, W1, b1.reshape(1, D), nd, ns)
    p2 = _agg_call(h1s, srcp, dstp)
    W2p = jnp.pad(W2, ((0, 0), (0, D - DOUT)))
    b2p = jnp.pad(b2, (0, D - DOUT)).reshape(1, D)
    out = _mm2_call(p2, W2p, b2p, nd)
    return out[:N, :DOUT]


# speedup vs baseline: 1.2743x; 1.2743x over previous
import functools

import jax
import jax.numpy as jnp
from jax import lax
from jax.experimental import pallas as pl
from jax.experimental.pallas import tpu as pltpu
from jax.experimental.pallas import tpu_sc as plsc

N = 10000
E = 320000
D = 128
DOUT = 121

NC, NS = 2, 16
NW = NC * NS
NPAD = 10240
EW = 10240
EPAD = NW * EW
CH = 128
NCH = EW // CH
WIN = 8
RPT = NPAD // NS
BIGC = 1
NCHB = 112
NCHS = 48
TCH = EPAD // CH
BR = 640


def _deg_body(srcp, dstp, ones_h, zeros_h, out, sidx, didx, ob, acc_s, acc_d):
    c = lax.axis_index("c")
    s = lax.axis_index("s")
    wid = c * NS + s
    pltpu.sync_copy(ones_h, ob)
    pltpu.sync_copy(zeros_h, acc_s.at[pl.ds(s * RPT, RPT)])
    pltpu.sync_copy(zeros_h, acc_d.at[pl.ds(s * RPT, RPT)])
    pltpu.sync_copy(srcp.at[pl.ds(wid * NCH, NCH)], sidx)
    pltpu.sync_copy(dstp.at[pl.ds(wid * NCH, NCH)], didx)
    plsc.subcore_barrier()

    def chunk(j, carry):
        pltpu.sync_copy(ob, acc_s.at[sidx.at[j]], add=True)
        pltpu.sync_copy(ob, acc_d.at[didx.at[j]], add=True)
        return carry

    lax.fori_loop(0, NCH, chunk, 0)
    plsc.subcore_barrier()
    pltpu.sync_copy(acc_s.at[pl.ds(s * RPT, RPT)],
                    out.at[c, 0, pl.ds(s * RPT, RPT)])
    pltpu.sync_copy(acc_d.at[pl.ds(s * RPT, RPT)],
                    out.at[c, 1, pl.ds(s * RPT, RPT)])


_deg_call = pl.kernel(
    _deg_body,
    out_type=jax.ShapeDtypeStruct((NC, 2, NPAD), jnp.float32),
    mesh=plsc.VectorSubcoreMesh(core_axis_name="c", subcore_axis_name="s"),
    scratch_types=[
        pltpu.VMEM((NCH, CH), jnp.int32),
        pltpu.VMEM((NCH, CH), jnp.int32),
        pltpu.VMEM((CH,), jnp.float32),
        pltpu.VMEM_SHARED((NPAD,), jnp.float32),
        pltpu.VMEM_SHARED((NPAD,), jnp.float32),
    ],
)


def _agg_body(xs, srcp, dstp, out, sbuf, dbuf, rows, acc, gsem, ssem):
    c = lax.axis_index("c")
    s = lax.axis_index("s")
    big = c == BIGC
    nch = jnp.where(big, NCHB, NCHS)
    cb = jnp.where(big, s * NCHB, jnp.minimum(NS * NCHB + s * NCHS, TCH - WIN))

    def zrow(r, carry):
        for k in range(D // 16):
            rows[0, r, pl.ds(k * 16, 16)] = jnp.zeros((16,), jnp.float32)
        return carry

    lax.fori_loop(0, CH, zrow, 0)
    for t in range(RPT // CH):
        pltpu.sync_copy(rows.at[0], acc.at[pl.ds(s * RPT + t * CH, CH)])
    plsc.subcore_barrier()

    def refill(half, w):
        pltpu.sync_copy(srcp.at[pl.ds(cb + w * WIN, WIN)], sbuf.at[half])
        pltpu.sync_copy(dstp.at[pl.ds(cb + w * WIN, WIN)], dbuf.at[half])

    def wait_gather(p):
        pltpu.make_async_copy(xs.at[sbuf.at[0, 0]], rows.at[p], gsem).wait()

    def wait_scatter(q):
        pltpu.make_async_copy(rows.at[q], acc.at[dbuf.at[0, 0]], ssem).wait()

    refill(0, 0)

    @pl.when(nch > 0)
    def _():
        pltpu.async_copy(xs.at[sbuf.at[0, 0]], rows.at[0], gsem)

    def body(i, carry):
        base0 = 2 * WIN * i
        @pl.when(i > 0)
        def _():
            wait_scatter(1)
        refill(1, 2 * i + 1)
        for j in range(2 * WIN):
            base = base0 + j
            p = j % 2
            q = 1 - p
            h, jj = (0, j) if j < WIN else (1, j - WIN)
            wait_gather(p)
            if j > 0:
                wait_scatter(q)
            if j == WIN:
                @pl.when(base0 + 2 * WIN < nch)
                def _():
                    refill(0, 2 * i + 2)
            nh, njj = (0, 0) if j == 2 * WIN - 1 else \
                      ((0, j + 1) if j + 1 < WIN else (1, j + 1 - WIN))

            @pl.when(base + 1 < nch)
            def _():
                pltpu.async_copy(xs.at[sbuf.at[nh, njj]], rows.at[q], gsem)

            pltpu.async_copy(rows.at[p], acc.at[dbuf.at[h, jj]], ssem,
                             add=True)
        return carry

    lax.fori_loop(0, nch // (2 * WIN), body, 0)

    @pl.when(nch > 0)
    def _():
        wait_scatter(1)

    plsc.subcore_barrier()
    pltpu.sync_copy(acc.at[pl.ds(s * RPT, RPT)],
                    out.at[c, pl.ds(s * RPT, RPT)])


_agg_call = pl.kernel(
    _agg_body,
    out_type=jax.ShapeDtypeStruct((NC, NPAD, D), jnp.float32),
    mesh=plsc.VectorSubcoreMesh(core_axis_name="c", subcore_axis_name="s"),
    scratch_types=[
        pltpu.VMEM((2, WIN, CH), jnp.int32),
        pltpu.VMEM((2, WIN, CH), jnp.int32),
        pltpu.VMEM((2, CH, D), jnp.float32),
        pltpu.VMEM_SHARED((NPAD, D), jnp.float32),
        pltpu.SemaphoreType.DMA,
        pltpu.SemaphoreType.DMA,
    ],
)


def _scale_body(feat_ref, cnt_ref, xs_ref, ns_ref, nd_ref):
    cnt = cnt_ref[...]
    cs = cnt[0, 0] + cnt[1, 0]
    cd = cnt[0, 1] + cnt[1, 1]
    ns = lax.rsqrt(jnp.maximum(cs, 1.0))
    nd = lax.rsqrt(jnp.maximum(cd, 1.0))
    xs_ref[...] = feat_ref[...] * ns
    ns_ref[...] = ns
    nd_ref[...] = nd


_scale_call = pl.pallas_call(
    _scale_body,
    grid=(NPAD // BR,),
    in_specs=[
        pl.BlockSpec((BR, D), lambda i: (i, 0)),
        pl.BlockSpec((NC, 2, BR, 1), lambda i: (0, 0, i, 0)),
    ],
    out_specs=[
        pl.BlockSpec((BR, D), lambda i: (i, 0)),
        pl.BlockSpec((BR, 1), lambda i: (i, 0)),
        pl.BlockSpec((BR, 1), lambda i: (i, 0)),
    ],
    out_shape=[
        jax.ShapeDtypeStruct((NPAD, D), jnp.float32),
        jax.ShapeDtypeStruct((NPAD, 1), jnp.float32),
        jax.ShapeDtypeStruct((NPAD, 1), jnp.float32),
    ],
)


def _mm1_body(p_ref, w_ref, b_ref, nd_ref, ns_ref, o_ref):
    a = p_ref[0] + p_ref[1]
    h = jnp.dot(a, w_ref[...], preferred_element_type=jnp.float32)
    h = h * nd_ref[...] + b_ref[...]
    o_ref[...] = jnp.maximum(h, 0.0) * ns_ref[...]


_mm1_call = pl.pallas_call(
    _mm1_body,
    grid=(NPAD // BR,),
    in_specs=[
        pl.BlockSpec((NC, BR, D), lambda i: (0, i, 0)),
        pl.BlockSpec((D, D), lambda i: (0, 0)),
        pl.BlockSpec((1, D), lambda i: (0, 0)),
        pl.BlockSpec((BR, 1), lambda i: (i, 0)),
        pl.BlockSpec((BR, 1), lambda i: (i, 0)),
    ],
    out_specs=pl.BlockSpec((BR, D), lambda i: (i, 0)),
    out_shape=jax.ShapeDtypeStruct((NPAD, D), jnp.float32),
)


def _mm2_body(p_ref, w_ref, b_ref, nd_ref, o_ref):
    a = p_ref[0] + p_ref[1]
    h = jnp.dot(a, w_ref[...], preferred_element_type=jnp.float32)
    h = h * nd_ref[...] + b_ref[...]
    o_ref[...] = jax.nn.sigmoid(h)


_mm2_call = pl.pallas_call(
    _mm2_body,
    grid=(NPAD // BR,),
    in_specs=[
        pl.BlockSpec((NC, BR, D), lambda i: (0, i, 0)),
        pl.BlockSpec((D, D), lambda i: (0, 0)),
        pl.BlockSpec((1, D), lambda i: (0, 0)),
        pl.BlockSpec((BR, 1), lambda i: (i, 0)),
    ],
    out_specs=pl.BlockSpec((BR, D), lambda i: (i, 0)),
    out_shape=jax.ShapeDtypeStruct((NPAD, D), jnp.float32),
)


@jax.jit
def kernel(feat, edge_index, W1, b1, W2, b2):
    src = edge_index[0].astype(jnp.int32)
    dst = edge_index[1].astype(jnp.int32)
    filler = jnp.full((EPAD - E,), N, jnp.int32)
    srcp = jnp.concatenate([src, filler]).reshape(TCH, CH)
    dstp = jnp.concatenate([dst, filler]).reshape(TCH, CH)
    featp = jnp.pad(feat, ((0, NPAD - N), (0, 0)))
    ones_h = jnp.ones((CH,), jnp.float32)
    zeros_h = jnp.zeros((RPT,), jnp.float32)

    cnt = _deg_call(srcp, dstp, ones_h, zeros_h).reshape(NC, 2, NPAD, 1)
    xs, ns, nd = _scale_call(featp, cnt)
    p1 = _agg_call(xs, srcp, dstp)
    h1s = _mm1_call(p1, W1, b1.reshape(1, D), nd, ns)
    p2 = _agg_call(h1s, srcp, dstp)
    W2p = jnp.pad(W2, ((0, 0), (0, D - DOUT)))
    b2p = jnp.pad(b2, (0, D - DOUT)).reshape(1, D)
    out = _mm2_call(p2, W2p, b2p, nd)
    return out[:N, :DOUT]


# speedup vs baseline: 1.5107x; 1.1855x over previous
import functools

import jax
import jax.numpy as jnp
from jax import lax
from jax.experimental import pallas as pl
from jax.experimental.pallas import tpu as pltpu
from jax.experimental.pallas import tpu_sc as plsc

N = 10000
E = 320000
D = 128
DOUT = 121

NC, NS = 2, 16
NW = NC * NS
NPAD = 10240
EW = 10240
EPAD = NW * EW
CH = 128
NCH = EW // CH
WIN = 8
RPT = NPAD // NS
BIGC = 1
NCHB = 144
NCHS = 16
TCH = EPAD // CH
BR = 640


def _deg_body(srcp, dstp, ones_h, zeros_h, out, sidx, didx, ob, acc_s, acc_d):
    c = lax.axis_index("c")
    s = lax.axis_index("s")
    wid = c * NS + s
    pltpu.sync_copy(ones_h, ob)
    pltpu.sync_copy(zeros_h, acc_s.at[pl.ds(s * RPT, RPT)])
    pltpu.sync_copy(zeros_h, acc_d.at[pl.ds(s * RPT, RPT)])
    pltpu.sync_copy(srcp.at[pl.ds(wid * NCH, NCH)], sidx)
    pltpu.sync_copy(dstp.at[pl.ds(wid * NCH, NCH)], didx)
    plsc.subcore_barrier()

    def chunk(j, carry):
        pltpu.sync_copy(ob, acc_s.at[sidx.at[j]], add=True)
        pltpu.sync_copy(ob, acc_d.at[didx.at[j]], add=True)
        return carry

    lax.fori_loop(0, NCH, chunk, 0)
    plsc.subcore_barrier()
    pltpu.sync_copy(acc_s.at[pl.ds(s * RPT, RPT)],
                    out.at[c, 0, pl.ds(s * RPT, RPT)])
    pltpu.sync_copy(acc_d.at[pl.ds(s * RPT, RPT)],
                    out.at[c, 1, pl.ds(s * RPT, RPT)])


_deg_call = pl.kernel(
    _deg_body,
    out_type=jax.ShapeDtypeStruct((NC, 2, NPAD), jnp.float32),
    mesh=plsc.VectorSubcoreMesh(core_axis_name="c", subcore_axis_name="s"),
    scratch_types=[
        pltpu.VMEM((NCH, CH), jnp.int32),
        pltpu.VMEM((NCH, CH), jnp.int32),
        pltpu.VMEM((CH,), jnp.float32),
        pltpu.VMEM_SHARED((NPAD,), jnp.float32),
        pltpu.VMEM_SHARED((NPAD,), jnp.float32),
    ],
)


def _agg_body(xs, srcp, dstp, out, sbuf, dbuf, rows, acc, gsem, ssem):
    c = lax.axis_index("c")
    s = lax.axis_index("s")
    big = c == BIGC
    nch = jnp.where(big, NCHB, NCHS)
    cb = jnp.where(big, s * NCHB, jnp.minimum(NS * NCHB + s * NCHS, TCH - WIN))

    def zrow(r, carry):
        for k in range(D // 16):
            rows[0, r, pl.ds(k * 16, 16)] = jnp.zeros((16,), jnp.float32)
        return carry

    lax.fori_loop(0, CH, zrow, 0)
    for t in range(RPT // CH):
        pltpu.sync_copy(rows.at[0], acc.at[pl.ds(s * RPT + t * CH, CH)])
    plsc.subcore_barrier()

    def refill(half, w):
        pltpu.sync_copy(srcp.at[pl.ds(cb + w * WIN, WIN)], sbuf.at[half])
        pltpu.sync_copy(dstp.at[pl.ds(cb + w * WIN, WIN)], dbuf.at[half])

    def wait_gather(p):
        pltpu.make_async_copy(xs.at[sbuf.at[0, 0]], rows.at[p], gsem).wait()

    def wait_scatter(q):
        pltpu.make_async_copy(rows.at[q], acc.at[dbuf.at[0, 0]], ssem).wait()

    refill(0, 0)

    @pl.when(nch > 0)
    def _():
        pltpu.async_copy(xs.at[sbuf.at[0, 0]], rows.at[0], gsem)

    def body(i, carry):
        base0 = 2 * WIN * i
        @pl.when(i > 0)
        def _():
            wait_scatter(1)
        refill(1, 2 * i + 1)
        for j in range(2 * WIN):
            base = base0 + j
            p = j % 2
            q = 1 - p
            h, jj = (0, j) if j < WIN else (1, j - WIN)
            wait_gather(p)
            if j > 0:
                wait_scatter(q)
            if j == WIN:
                @pl.when(base0 + 2 * WIN < nch)
                def _():
                    refill(0, 2 * i + 2)
            nh, njj = (0, 0) if j == 2 * WIN - 1 else \
                      ((0, j + 1) if j + 1 < WIN else (1, j + 1 - WIN))

            @pl.when(base + 1 < nch)
            def _():
                pltpu.async_copy(xs.at[sbuf.at[nh, njj]], rows.at[q], gsem)

            pltpu.async_copy(rows.at[p], acc.at[dbuf.at[h, jj]], ssem,
                             add=True)
        return carry

    lax.fori_loop(0, nch // (2 * WIN), body, 0)

    @pl.when(nch > 0)
    def _():
        wait_scatter(1)

    plsc.subcore_barrier()
    pltpu.sync_copy(acc.at[pl.ds(s * RPT, RPT)],
                    out.at[c, pl.ds(s * RPT, RPT)])


_agg_call = pl.kernel(
    _agg_body,
    out_type=jax.ShapeDtypeStruct((NC, NPAD, D), jnp.float32),
    mesh=plsc.VectorSubcoreMesh(core_axis_name="c", subcore_axis_name="s"),
    scratch_types=[
        pltpu.VMEM((2, WIN, CH), jnp.int32),
        pltpu.VMEM((2, WIN, CH), jnp.int32),
        pltpu.VMEM((2, CH, D), jnp.float32),
        pltpu.VMEM_SHARED((NPAD, D), jnp.float32),
        pltpu.SemaphoreType.DMA,
        pltpu.SemaphoreType.DMA,
    ],
)


def _scale_body(feat_ref, cnt_ref, xs_ref, ns_ref, nd_ref):
    cnt = cnt_ref[...]
    cs = cnt[0, 0] + cnt[1, 0]
    cd = cnt[0, 1] + cnt[1, 1]
    ns = lax.rsqrt(jnp.maximum(cs, 1.0))
    nd = lax.rsqrt(jnp.maximum(cd, 1.0))
    xs_ref[...] = feat_ref[...] * ns
    ns_ref[...] = ns
    nd_ref[...] = nd


_scale_call = pl.pallas_call(
    _scale_body,
    grid=(NPAD // BR,),
    in_specs=[
        pl.BlockSpec((BR, D), lambda i: (i, 0)),
        pl.BlockSpec((NC, 2, BR, 1), lambda i: (0, 0, i, 0)),
    ],
    out_specs=[
        pl.BlockSpec((BR, D), lambda i: (i, 0)),
        pl.BlockSpec((BR, 1), lambda i: (i, 0)),
        pl.BlockSpec((BR, 1), lambda i: (i, 0)),
    ],
    out_shape=[
        jax.ShapeDtypeStruct((NPAD, D), jnp.float32),
        jax.ShapeDtypeStruct((NPAD, 1), jnp.float32),
        jax.ShapeDtypeStruct((NPAD, 1), jnp.float32),
    ],
)


def _mm1_body(p_ref, w_ref, b_ref, nd_ref, ns_ref, o_ref):
    a = p_ref[0] + p_ref[1]
    h = jnp.dot(a, w_ref[...], preferred_element_type=jnp.float32)
    h = h * nd_ref[...] + b_ref[...]
    o_ref[...] = jnp.maximum(h, 0.0) * ns_ref[...]


_mm1_call = pl.pallas_call(
    _mm1_body,
    grid=(NPAD // BR,),
    in_specs=[
        pl.BlockSpec((NC, BR, D), lambda i: (0, i, 0)),
        pl.BlockSpec((D, D), lambda i: (0, 0)),
        pl.BlockSpec((1, D), lambda i: (0, 0)),
        pl.BlockSpec((BR, 1), lambda i: (i, 0)),
        pl.BlockSpec((BR, 1), lambda i: (i, 0)),
    ],
    out_specs=pl.BlockSpec((BR, D), lambda i: (i, 0)),
    out_shape=jax.ShapeDtypeStruct((NPAD, D), jnp.float32),
)


def _mm2_body(p_ref, w_ref, b_ref, nd_ref, o_ref):
    a = p_ref[0] + p_ref[1]
    h = jnp.dot(a, w_ref[...], preferred_element_type=jnp.float32)
    h = h * nd_ref[...] + b_ref[...]
    o_ref[...] = jax.nn.sigmoid(h)


_mm2_call = pl.pallas_call(
    _mm2_body,
    grid=(NPAD // BR,),
    in_specs=[
        pl.BlockSpec((NC, BR, D), lambda i: (0, i, 0)),
        pl.BlockSpec((D, D), lambda i: (0, 0)),
        pl.BlockSpec((1, D), lambda i: (0, 0)),
        pl.BlockSpec((BR, 1), lambda i: (i, 0)),
    ],
    out_specs=pl.BlockSpec((BR, D), lambda i: (i, 0)),
    out_shape=jax.ShapeDtypeStruct((NPAD, D), jnp.float32),
)


@jax.jit
def kernel(feat, edge_index, W1, b1, W2, b2):
    src = edge_index[0].astype(jnp.int32)
    dst = edge_index[1].astype(jnp.int32)
    filler = jnp.full((EPAD - E,), N, jnp.int32)
    srcp = jnp.concatenate([src, filler]).reshape(TCH, CH)
    dstp = jnp.concatenate([dst, filler]).reshape(TCH, CH)
    featp = jnp.pad(feat, ((0, NPAD - N), (0, 0)))
    ones_h = jnp.ones((CH,), jnp.float32)
    zeros_h = jnp.zeros((RPT,), jnp.float32)

    cnt = _deg_call(srcp, dstp, ones_h, zeros_h).reshape(NC, 2, NPAD, 1)
    xs, ns, nd = _scale_call(featp, cnt)
    p1 = _agg_call(xs, srcp, dstp)
    h1s = _mm1_call(p1, W1, b1.reshape(1, D), nd, ns)
    p2 = _agg_call(h1s, srcp, dstp)
    W2p = jnp.pad(W2, ((0, 0), (0, D - DOUT)))
    b2p = jnp.pad(b2, (0, D - DOUT)).reshape(1, D)
    out = _mm2_call(p2, W2p, b2p, nd)
    return out[:N, :DOUT]
